# baseline (speedup 1.0000x reference)
import functools

import jax
import jax.numpy as jnp
from jax import lax
from jax.experimental import pallas as pl
from jax.experimental.pallas import tpu as pltpu
from jax.experimental.pallas import tpu_sc as plsc

EMB = 64
SUB = 128
CHUNK = 512
GATHERS = CHUNK // SUB
NSLOTS = 2


def _embed_gather(idx2d, weight):
    B = idx2d.shape[0]
    info = plsc.get_sparse_core_info()
    nw = info.num_cores * info.num_subcores
    b_per_w = B // nw
    n_chunks = b_per_w // CHUNK

    mesh = plsc.VectorSubcoreMesh(core_axis_name="c", subcore_axis_name="s")

    @functools.partial(
        pl.kernel,
        mesh=mesh,
        compiler_params=pltpu.CompilerParams(use_tc_tiling_on_sc=False),
        out_type=jax.ShapeDtypeStruct((B, EMB), jnp.float32),
        scratch_types=[
            pltpu.VMEM((b_per_w,), jnp.int32),
            pltpu.VMEM((NSLOTS, CHUNK, EMB), jnp.float32),
            pltpu.SemaphoreType.DMA((NSLOTS,)),
            pltpu.SemaphoreType.DMA((NSLOTS,)),
        ],
    )
    def k(idx_hbm, w_hbm, out_hbm, idx_v, rows_v, sem_g, sem_w):
        wid = lax.axis_index("s") * info.num_cores + lax.axis_index("c")
        base = wid * b_per_w
        pltpu.sync_copy(idx_hbm.at[pl.ds(base, b_per_w)], idx_v)

        def g_copy(c, s):
            return pltpu.make_async_copy(
                w_hbm.at[idx_v.at[pl.ds(c * CHUNK, CHUNK)]],
                rows_v.at[s],
                sem_g.at[s],
            )

        def w_copy(c, s):
            return pltpu.make_async_copy(
                rows_v.at[s],
                out_hbm.at[pl.ds(base + c * CHUNK, CHUNK)],
                sem_w.at[s],
            )

        for s in range(NSLOTS):
            g_copy(s, s).start()

        def body(t, _):
            for s in range(NSLOTS):
                c = t * NSLOTS + s
                g_copy(c, s).wait()
                w_copy(c, s).start()
            for s in range(NSLOTS):
                c = t * NSLOTS + s
                w_copy(c, s).wait()
                g_copy(c + NSLOTS, s).start()
            return _

        lax.fori_loop(0, n_chunks // NSLOTS - 1, body, 0, unroll=False)

        for s in range(NSLOTS):
            c = n_chunks - NSLOTS + s
            g_copy(c, s).wait()
            w_copy(c, s).start()
        for s in range(NSLOTS):
            c = n_chunks - NSLOTS + s
            w_copy(c, s).wait()

    return k(idx2d, weight)


def kernel(input_, weight):
    S0, S1 = input_.shape
    out = _embed_gather(input_.reshape(S0 * S1), weight)
    return out.reshape(S0, S1, EMB)


# speedup vs baseline: 1.7440x; 1.7440x over previous
import functools

import jax
import jax.numpy as jnp
from jax import lax
from jax.experimental import pallas as pl
from jax.experimental.pallas import tpu as pltpu
from jax.experimental.pallas import tpu_sc as plsc

VOCAB = 128
EMB = 64
SUB = 128
CHUNK = 512
GATHERS = CHUNK // SUB
NSLOTS = 2


def _embed_gather(idx2d, weight):
    B = idx2d.shape[0]
    info = plsc.get_sparse_core_info()
    nw = info.num_cores * info.num_subcores
    b_per_w = B // nw
    n_chunks = b_per_w // CHUNK

    mesh = plsc.VectorSubcoreMesh(core_axis_name="c", subcore_axis_name="s")

    @functools.partial(
        pl.kernel,
        mesh=mesh,
        compiler_params=pltpu.CompilerParams(use_tc_tiling_on_sc=False),
        out_type=jax.ShapeDtypeStruct((B, EMB), jnp.float32),
        scratch_types=[
            pltpu.VMEM_SHARED((VOCAB, EMB), jnp.float32),
            pltpu.VMEM((b_per_w,), jnp.int32),
            pltpu.VMEM((NSLOTS, CHUNK, EMB), jnp.float32),
            pltpu.SemaphoreType.DMA((NSLOTS,)),
            pltpu.SemaphoreType.DMA((NSLOTS,)),
        ],
    )
    def k(idx_hbm, w_hbm, out_hbm, w_sp, idx_v, rows_v, sem_g, sem_w):
        wid = lax.axis_index("s") * info.num_cores + lax.axis_index("c")
        base = wid * b_per_w
        @pl.when(lax.axis_index("s") == 0)
        def _():
            pltpu.sync_copy(w_hbm, w_sp)
        pltpu.sync_copy(idx_hbm.at[pl.ds(base, b_per_w)], idx_v)
        plsc.subcore_barrier()

        def g_copy(c, s):
            return pltpu.make_async_copy(
                w_sp.at[idx_v.at[pl.ds(c * CHUNK, CHUNK)]],
                rows_v.at[s],
                sem_g.at[s],
            )

        def w_copy(c, s):
            return pltpu.make_async_copy(
                rows_v.at[s],
                out_hbm.at[pl.ds(base + c * CHUNK, CHUNK)],
                sem_w.at[s],
            )

        for s in range(NSLOTS):
            g_copy(s, s).start()

        def body(t, _):
            for s in range(NSLOTS):
                c = t * NSLOTS + s
                g_copy(c, s).wait()
                w_copy(c, s).start()
            for s in range(NSLOTS):
                c = t * NSLOTS + s
                w_copy(c, s).wait()
                g_copy(c + NSLOTS, s).start()
            return _

        lax.fori_loop(0, n_chunks // NSLOTS - 1, body, 0, unroll=False)

        for s in range(NSLOTS):
            c = n_chunks - NSLOTS + s
            g_copy(c, s).wait()
            w_copy(c, s).start()
        for s in range(NSLOTS):
            c = n_chunks - NSLOTS + s
            w_copy(c, s).wait()

    return k(idx2d, weight)


def kernel(input_, weight):
    S0, S1 = input_.shape
    out = _embed_gather(input_.reshape(S0 * S1), weight)
    return out.reshape(S0, S1, EMB)


# speedup vs baseline: 2.4736x; 1.4184x over previous
import functools

import jax
import jax.numpy as jnp
from jax import lax
from jax.experimental import pallas as pl
from jax.experimental.pallas import tpu as pltpu
from jax.experimental.pallas import tpu_sc as plsc

VOCAB = 128
EMB = 64
LANES = 128
CHUNK = 256
NSLOTS = 2


def _embed_gather(idx_flat, wpad):
    B = idx_flat.shape[0]
    info = plsc.get_sparse_core_info()
    nw = info.num_cores * info.num_subcores
    b_per_w = B // nw
    n_chunks = b_per_w // CHUNK

    mesh = plsc.VectorSubcoreMesh(core_axis_name="c", subcore_axis_name="s")

    @functools.partial(
        pl.kernel,
        mesh=mesh,
        compiler_params=pltpu.CompilerParams(use_tc_tiling_on_sc=False),
        out_type=jax.ShapeDtypeStruct((B, LANES), jnp.float32),
        scratch_types=[
            pltpu.VMEM_SHARED((VOCAB, LANES), jnp.float32),
            pltpu.VMEM((b_per_w,), jnp.int32),
            pltpu.VMEM((NSLOTS, CHUNK, LANES), jnp.float32),
            pltpu.SemaphoreType.DMA((NSLOTS,)),
            pltpu.SemaphoreType.DMA((NSLOTS,)),
        ],
    )
    def k(idx_hbm, w_hbm, out_hbm, w_sp, idx_v, rows_v, sem_g, sem_w):
        wid = lax.axis_index("s") * info.num_cores + lax.axis_index("c")
        base = wid * b_per_w
        @pl.when(lax.axis_index("s") == 0)
        def _():
            pltpu.sync_copy(w_hbm, w_sp)
        pltpu.sync_copy(idx_hbm.at[pl.ds(base, b_per_w)], idx_v)
        plsc.subcore_barrier()

        def g_copy(c, s):
            return pltpu.make_async_copy(
                w_sp.at[idx_v.at[pl.ds(c * CHUNK, CHUNK)]],
                rows_v.at[s],
                sem_g.at[s],
            )

        def w_copy(c, s):
            return pltpu.make_async_copy(
                rows_v.at[s],
                out_hbm.at[pl.ds(base + c * CHUNK, CHUNK)],
                sem_w.at[s],
            )

        for s in range(NSLOTS):
            g_copy(s, s).start()

        def body(t, _):
            for s in range(NSLOTS):
                c = t * NSLOTS + s
                g_copy(c, s).wait()
                w_copy(c, s).start()
            for s in range(NSLOTS):
                c = t * NSLOTS + s
                w_copy(c, s).wait()
                g_copy(c + NSLOTS, s).start()
            return _

        lax.fori_loop(0, n_chunks // NSLOTS - 1, body, 0, unroll=False)

        for s in range(NSLOTS):
            c = n_chunks - NSLOTS + s
            g_copy(c, s).wait()
            w_copy(c, s).start()
        for s in range(NSLOTS):
            c = n_chunks - NSLOTS + s
            w_copy(c, s).wait()

    return k(idx_flat, wpad)


def kernel(input_, weight):
    S0, S1 = input_.shape
    wpad = jnp.pad(weight, ((0, 0), (0, LANES - EMB)))
    out = _embed_gather(input_.reshape(S0 * S1), wpad)
    return out[:, :EMB].reshape(S0, S1, EMB)
